# baseline (speedup 1.0000x reference)
import jax
import jax.numpy as jnp
from jax.experimental import pallas as pl
from jax.experimental.pallas import tpu as pltpu


def _upsample_kernel(x_ref, m_ref, o_ref):
    wo = o_ref.shape[1]
    y = jnp.dot(
        x_ref[...], m_ref[...], preferred_element_type=jnp.float32
    ).astype(o_ref.dtype)
    ya = y[:, :wo]
    yb = y[:, wo:]
    o_ref[0::4, :] = ya
    o_ref[1::4, :] = ya
    o_ref[2::4, :] = yb
    o_ref[3::4, :] = yb


def _upsample2x(x2d, block_rows):
    rows, w2 = x2d.shape
    w_in = w2 // 2
    w_out = 2 * w_in
    grid = (rows // block_rows,)

    j = jnp.arange(w_out, dtype=jnp.int32)
    i = jnp.arange(w2, dtype=jnp.int32)[:, None]
    ma = (i == j[None, :] // 2)
    mb = (i == w_in + j[None, :] // 2)
    m = jnp.concatenate([ma, mb], axis=1).astype(jnp.float32)

    return pl.pallas_call(
        _upsample_kernel,
        out_shape=jax.ShapeDtypeStruct((4 * rows, w_out), x2d.dtype),
        grid_spec=pltpu.PrefetchScalarGridSpec(
            num_scalar_prefetch=0,
            grid=grid,
            in_specs=[
                pl.BlockSpec((block_rows, w2), lambda idx: (idx, 0)),
                pl.BlockSpec((w2, 2 * w_out), lambda idx: (0, 0)),
            ],
            out_specs=pl.BlockSpec((4 * block_rows, w_out), lambda idx: (idx, 0)),
        ),
        compiler_params=pltpu.CompilerParams(
            dimension_semantics=("parallel",),
            vmem_limit_bytes=64 * 1024 * 1024,
        ),
        cost_estimate=pl.CostEstimate(
            flops=2 * rows * w2 * 2 * w_out,
            transcendentals=0,
            bytes_accessed=5 * rows * w2 * x2d.dtype.itemsize,
        ),
    )(x2d, m)


@jax.jit
def kernel(x):
    b, c, h, w = x.shape
    x2d = x.reshape(b * c * h * w // 128, 128)
    out2d = _upsample2x(x2d, block_rows=8192)
    return out2d.reshape(b, c, 2 * h, 2 * w)


# speedup vs baseline: 1.6590x; 1.6590x over previous
import jax
import jax.numpy as jnp
from jax.experimental import pallas as pl
from jax.experimental.pallas import tpu as pltpu


def _upsample_kernel(x_ref, ct_ref, o_ref):
    y = jnp.dot(
        x_ref[...], ct_ref[...], preferred_element_type=jnp.float32
    ).astype(o_ref.dtype)
    o_ref[::2, :] = y
    o_ref[1::2, :] = y


def _upsample2x_rows(x2d, block_rows):
    rows, w_in = x2d.shape
    w_out = 2 * w_in
    grid = (rows // block_rows,)

    col_src = jnp.arange(w_out, dtype=jnp.int32) // 2
    ct = (jnp.arange(w_in, dtype=jnp.int32)[:, None] == col_src[None, :])
    ct = ct.astype(jnp.float32)

    return pl.pallas_call(
        _upsample_kernel,
        out_shape=jax.ShapeDtypeStruct((2 * rows, w_out), x2d.dtype),
        grid_spec=pltpu.PrefetchScalarGridSpec(
            num_scalar_prefetch=0,
            grid=grid,
            in_specs=[
                pl.BlockSpec((block_rows, w_in), lambda i: (i, 0)),
                pl.BlockSpec((w_in, w_out), lambda i: (0, 0)),
            ],
            out_specs=pl.BlockSpec((2 * block_rows, w_out), lambda i: (i, 0)),
        ),
        compiler_params=pltpu.CompilerParams(
            dimension_semantics=("parallel",),
            vmem_limit_bytes=64 * 1024 * 1024,
        ),
        cost_estimate=pl.CostEstimate(
            flops=2 * rows * w_in * w_out,
            transcendentals=0,
            bytes_accessed=rows * (w_in + 4 * w_in) * x2d.dtype.itemsize,
        ),
    )(x2d, ct)


@jax.jit
def kernel(x):
    b, c, h, w = x.shape
    x2d = x.reshape(b * c * h, w)
    out2d = _upsample2x_rows(x2d, block_rows=16384)
    return out2d.reshape(b, c, 2 * h, 2 * w)
